# baseline (speedup 1.0000x reference)
import functools

import jax
import jax.numpy as jnp
from jax import lax
from jax.experimental import pallas as pl
from jax.experimental.pallas import tpu as pltpu
from jax.experimental.pallas import tpu_sc as plsc

_G = 8
_EPS = 1e-5
_BLK = 8192
_SUB = 128


def _stats_tc_kernel(x_ref, idx_ref, out_ref):
    x = x_ref[...]
    idx = idx_ref[0, 0, :]
    nb, c = x.shape
    ni = out_ref.shape[0]
    onehot = (idx[:, None] == lax.broadcasted_iota(jnp.int32, (nb, ni), 1)
              ).astype(jnp.float32)
    buf = jnp.concatenate(
        [x, x * x, jnp.ones((nb, 16), jnp.float32)], axis=1)

    @pl.when(pl.program_id(0) == 0)
    def _():
        out_ref[...] = jnp.zeros_like(out_ref)

    out_ref[...] += lax.dot_general(
        onehot, buf, (((0,), (0,)), ((), ())),
        preferred_element_type=jnp.float32)


def _finalize_tc_kernel(st_ref, wb_ref, ab_ref):
    ni = ab_ref.shape[0]
    c = ab_ref.shape[1] // 2
    st = st_ref[...]
    sumx = st[:, :c]
    sumsq = st[:, c:2 * c]
    cnt = st[:, 2 * c:2 * c + 1]
    cpg = c // _G

    denom = jnp.maximum(cnt, 1.0) * cpg
    gsel = (lax.broadcasted_iota(jnp.int32, (c, _G), 0) // cpg
            == lax.broadcasted_iota(jnp.int32, (c, _G), 1)).astype(jnp.float32)
    sum_g = jnp.dot(sumx, gsel, preferred_element_type=jnp.float32)
    sq_g = jnp.dot(sumsq, gsel, preferred_element_type=jnp.float32)
    mean_g = sum_g / denom
    var_g = sq_g / denom - mean_g * mean_g
    rstd_g = lax.rsqrt(var_g + _EPS)
    mean_c = lax.dot_general(mean_g, gsel, (((1,), (1,)), ((), ())),
                             preferred_element_type=jnp.float32)
    rstd_c = lax.dot_general(rstd_g, gsel, (((1,), (1,)), ((), ())),
                             preferred_element_type=jnp.float32)
    w = wb_ref[0:1, :]
    b = wb_ref[1:2, :]
    a_coef = rstd_c * w
    b_coef = b - mean_c * a_coef
    ab_ref[...] = jnp.concatenate([a_coef, b_coef], axis=1)


def _apply_sc_kernel(nc, ns, chunks,
                     feat_hbm, idx_hbm, ab_hbm, out_hbm,
                     xb0, xb1, ab0, ab1, iball, tbuf, absh,
                     dsem0, dsem1, gsem0, gsem1, osem0, osem1):
    cid = lax.axis_index("c")
    sid = lax.axis_index("s")
    wid = sid * nc + cid
    base_chunk = wid * chunks

    pltpu.sync_copy(idx_hbm.at[pl.ds(base_chunk, chunks)], iball)

    @pl.when(sid == 0)
    def _():
        pltpu.sync_copy(ab_hbm, tbuf)
        pltpu.sync_copy(tbuf, absh)
    plsc.subcore_barrier()

    def dma_in(k, xb, sem):
        return pltpu.make_async_copy(
            feat_hbm.at[pl.ds((base_chunk + k) * _SUB, _SUB)], xb, sem)

    def dma_out(k, xb, sem):
        return pltpu.make_async_copy(
            xb, out_hbm.at[pl.ds((base_chunk + k) * _SUB, _SUB)], sem)

    def gather(k, ab, sem):
        return pltpu.make_async_copy(absh.at[iball.at[k]], ab, sem)

    def stage(k, xb, ab, dsem, gsem, osem, pxb, pab, pdsem, pgsem, posem):
        @pl.when(k + 1 < chunks)
        def _():
            gather(k + 1, pab, pgsem).start()

        dma_in(k, xb, dsem).wait()
        gather(k, ab, gsem).wait()

        @plsc.parallel_loop(0, _SUB, unroll=4)
        def _(r):
            for j in range(4):
                v = xb[r, pl.ds(16 * j, 16)]
                a = ab[r, pl.ds(16 * j, 16)]
                b = ab[r, pl.ds(64 + 16 * j, 16)]
                xb[r, pl.ds(16 * j, 16)] = v * a + b

        dma_out(k, xb, osem).start()

        @pl.when(k >= 1)
        def _():
            dma_out(k - 1, pxb, posem).wait()

        @pl.when(k + 1 < chunks)
        def _():
            dma_in(k + 1, pxb, pdsem).start()

    dma_in(0, xb0, dsem0).start()
    gather(0, ab0, gsem0).start()

    def pair(kk, carry):
        k0 = 2 * kk
        stage(k0, xb0, ab0, dsem0, gsem0, osem0,
              xb1, ab1, dsem1, gsem1, osem1)
        stage(k0 + 1, xb1, ab1, dsem1, gsem1, osem1,
              xb0, ab0, dsem0, gsem0, osem0)
        return carry
    lax.fori_loop(0, chunks // 2, pair, 0)

    dma_out(chunks - 1, xb1, osem1).wait()


def kernel(features, ins_indices_batch, ins_ids, ins_indices_len, weight, bias):
    n, c = features.shape
    ni = ins_ids.shape[0]
    nblk = n // _BLK
    idx3 = ins_indices_batch.reshape(nblk, 1, _BLK)
    idx2 = ins_indices_batch.reshape(n // _SUB, _SUB)
    sc = 2 * c + 16

    stats = pl.pallas_call(
        _stats_tc_kernel,
        grid=(nblk,),
        in_specs=[
            pl.BlockSpec((_BLK, c), lambda i: (i, 0)),
            pl.BlockSpec((1, 1, _BLK), lambda i: (i, 0, 0)),
        ],
        out_specs=pl.BlockSpec((ni, sc), lambda i: (0, 0)),
        out_shape=jax.ShapeDtypeStruct((ni, sc), jnp.float32),
        compiler_params=pltpu.CompilerParams(
            dimension_semantics=("arbitrary",)),
    )(features, idx3)

    wb = jnp.stack([weight, bias], axis=0)

    ab = pl.pallas_call(
        _finalize_tc_kernel,
        grid=(1,),
        in_specs=[
            pl.BlockSpec((ni, sc), lambda i: (0, 0)),
            pl.BlockSpec((2, c), lambda i: (0, 0)),
        ],
        out_specs=pl.BlockSpec((ni, 2 * c), lambda i: (0, 0)),
        out_shape=jax.ShapeDtypeStruct((ni, 2 * c), jnp.float32),
    )(stats, wb)

    info = plsc.get_sparse_core_info()
    nc, ns = info.num_cores, info.num_subcores
    nw = nc * ns
    chunks2 = n // (nw * _SUB)
    apply_fn = functools.partial(
        pl.kernel,
        mesh=plsc.VectorSubcoreMesh(core_axis_name="c", subcore_axis_name="s"),
        out_type=jax.ShapeDtypeStruct((n, c), jnp.float32),
        scratch_types=[
            pltpu.VMEM((_SUB, c), jnp.float32),
            pltpu.VMEM((_SUB, c), jnp.float32),
            pltpu.VMEM((_SUB, 2 * c), jnp.float32),
            pltpu.VMEM((_SUB, 2 * c), jnp.float32),
            pltpu.VMEM((chunks2, _SUB), jnp.int32),
            pltpu.VMEM((ni, 2 * c), jnp.float32),
            pltpu.VMEM_SHARED((ni, 2 * c), jnp.float32),
            pltpu.SemaphoreType.DMA,
            pltpu.SemaphoreType.DMA,
            pltpu.SemaphoreType.DMA,
            pltpu.SemaphoreType.DMA,
            pltpu.SemaphoreType.DMA,
            pltpu.SemaphoreType.DMA,
        ],
    )(functools.partial(_apply_sc_kernel, nc, ns, chunks2))
    out = apply_fn(features, idx2, ab)
    return out


# speedup vs baseline: 1.0010x; 1.0010x over previous
import functools

import jax
import jax.numpy as jnp
from jax import lax
from jax.experimental import pallas as pl
from jax.experimental.pallas import tpu as pltpu
from jax.experimental.pallas import tpu_sc as plsc

_G = 8
_EPS = 1e-5
_BLK = 8192
_SUB = 128


def _stats_tc_kernel(x_ref, idx_ref, wb_ref, st_ref, ab_ref):
    x = x_ref[...]
    idx = idx_ref[0, 0, :]
    nb, c = x.shape
    ni = st_ref.shape[0]
    onehot = (idx[:, None] == lax.broadcasted_iota(jnp.int32, (nb, ni), 1)
              ).astype(jnp.float32)
    buf = jnp.concatenate(
        [x, x * x, jnp.ones((nb, 16), jnp.float32)], axis=1)

    @pl.when(pl.program_id(0) == 0)
    def _():
        st_ref[...] = jnp.zeros_like(st_ref)

    st_ref[...] += lax.dot_general(
        onehot, buf, (((0,), (0,)), ((), ())),
        preferred_element_type=jnp.float32)

    @pl.when(pl.program_id(0) == pl.num_programs(0) - 1)
    def _():
        st = st_ref[...]
        sumx = st[:, :c]
        sumsq = st[:, c:2 * c]
        cnt = st[:, 2 * c:2 * c + 1]
        cpg = c // _G
        denom = jnp.maximum(cnt, 1.0) * cpg
        gsel = (lax.broadcasted_iota(jnp.int32, (c, _G), 0) // cpg
                == lax.broadcasted_iota(jnp.int32, (c, _G), 1)
                ).astype(jnp.float32)
        sum_g = jnp.dot(sumx, gsel, preferred_element_type=jnp.float32)
        sq_g = jnp.dot(sumsq, gsel, preferred_element_type=jnp.float32)
        mean_g = sum_g / denom
        var_g = sq_g / denom - mean_g * mean_g
        rstd_g = lax.rsqrt(var_g + _EPS)
        mean_c = lax.dot_general(mean_g, gsel, (((1,), (1,)), ((), ())),
                                 preferred_element_type=jnp.float32)
        rstd_c = lax.dot_general(rstd_g, gsel, (((1,), (1,)), ((), ())),
                                 preferred_element_type=jnp.float32)
        w = wb_ref[0:1, :]
        b = wb_ref[1:2, :]
        a_coef = rstd_c * w
        b_coef = b - mean_c * a_coef
        ab_ref[...] = jnp.concatenate([a_coef, b_coef], axis=1)


def _apply_sc_kernel(nc, ns, chunks,
                     feat_hbm, idx_hbm, ab_hbm, out_hbm,
                     xb0, xb1, ab0, ab1, iball, tbuf, absh,
                     dsem0, dsem1, gsem0, gsem1, osem0, osem1):
    cid = lax.axis_index("c")
    sid = lax.axis_index("s")
    wid = sid * nc + cid
    base_chunk = wid * chunks

    pltpu.sync_copy(idx_hbm.at[pl.ds(base_chunk, chunks)], iball)

    @pl.when(sid == 0)
    def _():
        pltpu.sync_copy(ab_hbm, tbuf)
        pltpu.sync_copy(tbuf, absh)
    plsc.subcore_barrier()

    def dma_in(k, xb, sem):
        return pltpu.make_async_copy(
            feat_hbm.at[pl.ds((base_chunk + k) * _SUB, _SUB)], xb, sem)

    def dma_out(k, xb, sem):
        return pltpu.make_async_copy(
            xb, out_hbm.at[pl.ds((base_chunk + k) * _SUB, _SUB)], sem)

    def gather(k, ab, sem):
        return pltpu.make_async_copy(absh.at[iball.at[k]], ab, sem)

    def stage(k, xb, ab, dsem, gsem, osem, pxb, pab, pdsem, pgsem, posem):
        @pl.when(k + 1 < chunks)
        def _():
            gather(k + 1, pab, pgsem).start()

        dma_in(k, xb, dsem).wait()
        gather(k, ab, gsem).wait()

        @plsc.parallel_loop(0, _SUB, unroll=4)
        def _(r):
            for j in range(4):
                v = xb[r, pl.ds(16 * j, 16)]
                a = ab[r, pl.ds(16 * j, 16)]
                b = ab[r, pl.ds(64 + 16 * j, 16)]
                xb[r, pl.ds(16 * j, 16)] = v * a + b

        dma_out(k, xb, osem).start()

        @pl.when(k >= 1)
        def _():
            dma_out(k - 1, pxb, posem).wait()

        @pl.when(k + 1 < chunks)
        def _():
            dma_in(k + 1, pxb, pdsem).start()

    dma_in(0, xb0, dsem0).start()
    gather(0, ab0, gsem0).start()

    def pair(kk, carry):
        k0 = 2 * kk
        stage(k0, xb0, ab0, dsem0, gsem0, osem0,
              xb1, ab1, dsem1, gsem1, osem1)
        stage(k0 + 1, xb1, ab1, dsem1, gsem1, osem1,
              xb0, ab0, dsem0, gsem0, osem0)
        return carry
    lax.fori_loop(0, chunks // 2, pair, 0)

    dma_out(chunks - 1, xb1, osem1).wait()


def kernel(features, ins_indices_batch, ins_ids, ins_indices_len, weight, bias):
    n, c = features.shape
    ni = ins_ids.shape[0]
    nblk = n // _BLK
    idx3 = ins_indices_batch.reshape(nblk, 1, _BLK)
    idx2 = ins_indices_batch.reshape(n // _SUB, _SUB)
    sc = 2 * c + 16

    wb = jnp.stack([weight, bias], axis=0)

    _, ab = pl.pallas_call(
        _stats_tc_kernel,
        grid=(nblk,),
        in_specs=[
            pl.BlockSpec((_BLK, c), lambda i: (i, 0)),
            pl.BlockSpec((1, 1, _BLK), lambda i: (i, 0, 0)),
            pl.BlockSpec((2, c), lambda i: (0, 0)),
        ],
        out_specs=[
            pl.BlockSpec((ni, sc), lambda i: (0, 0)),
            pl.BlockSpec((ni, 2 * c), lambda i: (0, 0)),
        ],
        out_shape=[
            jax.ShapeDtypeStruct((ni, sc), jnp.float32),
            jax.ShapeDtypeStruct((ni, 2 * c), jnp.float32),
        ],
        compiler_params=pltpu.CompilerParams(
            dimension_semantics=("arbitrary",)),
    )(features, idx3, wb)

    info = plsc.get_sparse_core_info()
    nc, ns = info.num_cores, info.num_subcores
    nw = nc * ns
    chunks2 = n // (nw * _SUB)
    apply_fn = functools.partial(
        pl.kernel,
        mesh=plsc.VectorSubcoreMesh(core_axis_name="c", subcore_axis_name="s"),
        out_type=jax.ShapeDtypeStruct((n, c), jnp.float32),
        scratch_types=[
            pltpu.VMEM((_SUB, c), jnp.float32),
            pltpu.VMEM((_SUB, c), jnp.float32),
            pltpu.VMEM((_SUB, 2 * c), jnp.float32),
            pltpu.VMEM((_SUB, 2 * c), jnp.float32),
            pltpu.VMEM((chunks2, _SUB), jnp.int32),
            pltpu.VMEM((ni, 2 * c), jnp.float32),
            pltpu.VMEM_SHARED((ni, 2 * c), jnp.float32),
            pltpu.SemaphoreType.DMA,
            pltpu.SemaphoreType.DMA,
            pltpu.SemaphoreType.DMA,
            pltpu.SemaphoreType.DMA,
            pltpu.SemaphoreType.DMA,
            pltpu.SemaphoreType.DMA,
        ],
    )(functools.partial(_apply_sc_kernel, nc, ns, chunks2))
    out = apply_fn(features, idx2, ab)
    return out


# speedup vs baseline: 1.0068x; 1.0058x over previous
import functools

import jax
import jax.numpy as jnp
from jax import lax
from jax.experimental import pallas as pl
from jax.experimental.pallas import tpu as pltpu
from jax.experimental.pallas import tpu_sc as plsc

_G = 8
_EPS = 1e-5
_BLK = 16384
_SUB = 128


def _stats_tc_kernel(x_ref, idx_ref, wb_ref, st_ref, ab_ref):
    x = x_ref[...]
    idx = idx_ref[0, 0, :]
    nb, c = x.shape
    ni = st_ref.shape[0]
    onehot = (idx[:, None] == lax.broadcasted_iota(jnp.int32, (nb, ni), 1)
              ).astype(jnp.float32)
    buf = jnp.concatenate(
        [x, x * x, jnp.ones((nb, 16), jnp.float32)], axis=1)

    @pl.when(pl.program_id(0) == 0)
    def _():
        st_ref[...] = jnp.zeros_like(st_ref)

    st_ref[...] += lax.dot_general(
        onehot, buf, (((0,), (0,)), ((), ())),
        preferred_element_type=jnp.float32)

    @pl.when(pl.program_id(0) == pl.num_programs(0) - 1)
    def _():
        st = st_ref[...]
        sumx = st[:, :c]
        sumsq = st[:, c:2 * c]
        cnt = st[:, 2 * c:2 * c + 1]
        cpg = c // _G
        denom = jnp.maximum(cnt, 1.0) * cpg
        gsel = (lax.broadcasted_iota(jnp.int32, (c, _G), 0) // cpg
                == lax.broadcasted_iota(jnp.int32, (c, _G), 1)
                ).astype(jnp.float32)
        sum_g = jnp.dot(sumx, gsel, preferred_element_type=jnp.float32)
        sq_g = jnp.dot(sumsq, gsel, preferred_element_type=jnp.float32)
        mean_g = sum_g / denom
        var_g = sq_g / denom - mean_g * mean_g
        rstd_g = lax.rsqrt(var_g + _EPS)
        mean_c = lax.dot_general(mean_g, gsel, (((1,), (1,)), ((), ())),
                                 preferred_element_type=jnp.float32)
        rstd_c = lax.dot_general(rstd_g, gsel, (((1,), (1,)), ((), ())),
                                 preferred_element_type=jnp.float32)
        w = wb_ref[0:1, :]
        b = wb_ref[1:2, :]
        a_coef = rstd_c * w
        b_coef = b - mean_c * a_coef
        ab_ref[...] = jnp.concatenate([a_coef, b_coef], axis=1)


def _apply_sc_kernel(nc, ns, chunks,
                     feat_hbm, idx_hbm, ab_hbm, out_hbm,
                     xb0, xb1, ab0, ab1, iball, tbuf, absh,
                     dsem0, dsem1, gsem0, gsem1, osem0, osem1):
    cid = lax.axis_index("c")
    sid = lax.axis_index("s")
    wid = sid * nc + cid
    base_chunk = wid * chunks

    pltpu.sync_copy(idx_hbm.at[pl.ds(base_chunk, chunks)], iball)

    @pl.when(sid == 0)
    def _():
        pltpu.sync_copy(ab_hbm, tbuf)
        pltpu.sync_copy(tbuf, absh)
    plsc.subcore_barrier()

    def dma_in(k, xb, sem):
        return pltpu.make_async_copy(
            feat_hbm.at[pl.ds((base_chunk + k) * _SUB, _SUB)], xb, sem)

    def dma_out(k, xb, sem):
        return pltpu.make_async_copy(
            xb, out_hbm.at[pl.ds((base_chunk + k) * _SUB, _SUB)], sem)

    def gather(k, ab, sem):
        return pltpu.make_async_copy(absh.at[iball.at[k]], ab, sem)

    def stage(k, xb, ab, dsem, gsem, osem, pxb, pab, pdsem, pgsem, posem):
        @pl.when(k + 1 < chunks)
        def _():
            gather(k + 1, pab, pgsem).start()

        dma_in(k, xb, dsem).wait()
        gather(k, ab, gsem).wait()

        @plsc.parallel_loop(0, _SUB, unroll=4)
        def _(r):
            for j in range(4):
                v = xb[r, pl.ds(16 * j, 16)]
                a = ab[r, pl.ds(16 * j, 16)]
                b = ab[r, pl.ds(64 + 16 * j, 16)]
                xb[r, pl.ds(16 * j, 16)] = v * a + b

        dma_out(k, xb, osem).start()

        @pl.when(k >= 1)
        def _():
            dma_out(k - 1, pxb, posem).wait()

        @pl.when(k + 1 < chunks)
        def _():
            dma_in(k + 1, pxb, pdsem).start()

    dma_in(0, xb0, dsem0).start()
    gather(0, ab0, gsem0).start()

    def pair(kk, carry):
        k0 = 2 * kk
        stage(k0, xb0, ab0, dsem0, gsem0, osem0,
              xb1, ab1, dsem1, gsem1, osem1)
        stage(k0 + 1, xb1, ab1, dsem1, gsem1, osem1,
              xb0, ab0, dsem0, gsem0, osem0)
        return carry
    lax.fori_loop(0, chunks // 2, pair, 0)

    dma_out(chunks - 1, xb1, osem1).wait()


def kernel(features, ins_indices_batch, ins_ids, ins_indices_len, weight, bias):
    n, c = features.shape
    ni = ins_ids.shape[0]
    nblk = n // _BLK
    idx3 = ins_indices_batch.reshape(nblk, 1, _BLK)
    idx2 = ins_indices_batch.reshape(n // _SUB, _SUB)
    sc = 2 * c + 16

    wb = jnp.stack([weight, bias], axis=0)

    _, ab = pl.pallas_call(
        _stats_tc_kernel,
        grid=(nblk,),
        in_specs=[
            pl.BlockSpec((_BLK, c), lambda i: (i, 0)),
            pl.BlockSpec((1, 1, _BLK), lambda i: (i, 0, 0)),
            pl.BlockSpec((2, c), lambda i: (0, 0)),
        ],
        out_specs=[
            pl.BlockSpec((ni, sc), lambda i: (0, 0)),
            pl.BlockSpec((ni, 2 * c), lambda i: (0, 0)),
        ],
        out_shape=[
            jax.ShapeDtypeStruct((ni, sc), jnp.float32),
            jax.ShapeDtypeStruct((ni, 2 * c), jnp.float32),
        ],
        compiler_params=pltpu.CompilerParams(
            dimension_semantics=("arbitrary",)),
    )(features, idx3, wb)

    info = plsc.get_sparse_core_info()
    nc, ns = info.num_cores, info.num_subcores
    nw = nc * ns
    chunks2 = n // (nw * _SUB)
    apply_fn = functools.partial(
        pl.kernel,
        mesh=plsc.VectorSubcoreMesh(core_axis_name="c", subcore_axis_name="s"),
        out_type=jax.ShapeDtypeStruct((n, c), jnp.float32),
        scratch_types=[
            pltpu.VMEM((_SUB, c), jnp.float32),
            pltpu.VMEM((_SUB, c), jnp.float32),
            pltpu.VMEM((_SUB, 2 * c), jnp.float32),
            pltpu.VMEM((_SUB, 2 * c), jnp.float32),
            pltpu.VMEM((chunks2, _SUB), jnp.int32),
            pltpu.VMEM((ni, 2 * c), jnp.float32),
            pltpu.VMEM_SHARED((ni, 2 * c), jnp.float32),
            pltpu.SemaphoreType.DMA,
            pltpu.SemaphoreType.DMA,
            pltpu.SemaphoreType.DMA,
            pltpu.SemaphoreType.DMA,
            pltpu.SemaphoreType.DMA,
            pltpu.SemaphoreType.DMA,
        ],
    )(functools.partial(_apply_sc_kernel, nc, ns, chunks2))
    out = apply_fn(features, idx2, ab)
    return out
